# baseline (speedup 1.0000x reference)
import functools

import numpy as np
import jax
import jax.numpy as jnp
from jax.experimental import pallas as pl
from jax.experimental.pallas import tpu as pltpu

B = 8
IN_FEATS = 256
HID = 512
NUM_CLASSES = 1000
NUM_NODES = 16
NODE_DIM = IN_FEATS // NUM_NODES
H = W = 224
KPAD = 32


def _conv_mean_body(x_ref, w_ref, o_ref):
    zrows = jnp.zeros((2, 256), jnp.bfloat16)

    def block_step(i, acc):
        y0 = pl.multiple_of(i * 8, 8)
        win = [x_ref[0, j, pl.ds(y0, 16), :] for j in range(10)]
        for r in range(8):
            pt = jnp.concatenate([wj[r:r + 3] for wj in win] + [zrows],
                                 axis=0)
            z = jax.lax.dot_general(
                pt, w_ref[...],
                dimension_numbers=(((0,), (0,)), ((), ())),
                preferred_element_type=jnp.float32)
            acc = acc + jnp.sum(
                jnp.maximum(z, 0.0).reshape(32, 8, IN_FEATS), axis=0)
        return acc

    acc = jax.lax.fori_loop(0, H // 8, block_step,
                            jnp.zeros((8, IN_FEATS), jnp.float32))
    o_ref[0, 0, :] = jnp.sum(acc, axis=0) * jnp.float32(1.0 / (H * W))


def _gcn_layer1_body(agg_ref, d_ref, w1t_ref, b1_ref, o_ref):
    f32 = jnp.float32
    agg = agg_ref[...] * d_ref[...]
    o_ref[...] = jnp.maximum(
        jax.lax.dot_general(agg, w1t_ref[...], (((1,), (0,)), ((), ())),
                            preferred_element_type=f32) + b1_ref[...], 0.0)


def _gcn_layer2_body(agg_ref, d_ref, p_ref, w2t_ref, b2_ref,
                     wfct_ref, bfc_ref, o_ref):
    f32 = jnp.float32
    agg = agg_ref[...] * d_ref[...]
    h2 = jnp.maximum(
        jax.lax.dot_general(agg, w2t_ref[...], (((1,), (0,)), ((), ())),
                            preferred_element_type=f32) + b2_ref[...], 0.0)
    pooled = jax.lax.dot_general(
        p_ref[...], h2, (((1,), (0,)), ((), ())), preferred_element_type=f32)
    o_ref[...] = jax.lax.dot_general(
        pooled, wfct_ref[...], (((1,), (0,)), ((), ())),
        preferred_element_type=f32) + bfc_ref[...]


def _deg_vector():
    d = np.array([1.0 / max(NUM_NODES - 1 - i, 1) for i in range(NUM_NODES)],
                 np.float32)
    return jnp.asarray(np.tile(d, B)[:, None])


def _pool_matrix():
    p = np.kron(np.eye(B, dtype=np.float32),
                np.full((1, NUM_NODES), 1.0 / NUM_NODES, np.float32))
    return jnp.asarray(p)


def _sc_segment_suffix_sum(data, chunk_w):
    from jax.experimental.pallas import tpu_sc as plsc
    nrows, d_feat = data.shape
    d_chunks = d_feat // chunk_w
    ntasks = B * d_chunks
    mesh = plsc.VectorSubcoreMesh(core_axis_name="c", subcore_axis_name="s")

    @functools.partial(
        pl.kernel,
        out_type=jax.ShapeDtypeStruct((nrows, d_feat), jnp.float32),
        mesh=mesh,
        scratch_types=[pltpu.VMEM((16, NUM_NODES, chunk_w), jnp.float32),
                       pltpu.VMEM((16, NUM_NODES, chunk_w), jnp.float32),
                       pltpu.SemaphoreType.DMA((16,))],
    )
    def agg_kernel(x_hbm, o_hbm, in_vs, out_vs, sems):
        sidx = jax.lax.axis_index("s")
        t = jax.lax.axis_index("c") * 16 + sidx
        in_v = in_vs.at[sidx]
        out_v = out_vs.at[sidx]
        sem = sems.at[sidx]

        @pl.when(t < ntasks)
        def _():
            bq = t // d_chunks
            cq = jax.lax.rem(t, d_chunks)
            r0 = bq * NUM_NODES
            c0 = cq * chunk_w
            pltpu.async_copy(
                x_hbm.at[pl.ds(r0, NUM_NODES), pl.ds(c0, chunk_w)],
                in_v, sem).wait()

            @pl.loop(0, chunk_w, step=16)
            def _(l):
                sl = pl.ds(l, 16)
                out_v.at[NUM_NODES - 1, sl][...] = jnp.zeros((16,), jnp.float32)

                @pl.loop(0, NUM_NODES - 1)
                def _(ii):
                    i = NUM_NODES - 2 - ii
                    up = in_v.at[i + 1, sl][...]
                    out_v.at[i, sl][...] = up
                    in_v.at[i, sl][...] = in_v.at[i, sl][...] + up

            pltpu.async_copy(
                out_v, o_hbm.at[pl.ds(r0, NUM_NODES), pl.ds(c0, chunk_w)],
                sem).wait()

    return agg_kernel(data)


def kernel(x, conv_w, conv_b, w1, b1, w2, b2, wfc, bfc):
    xp = jnp.pad(x, ((0, 0), (0, 0), (1, 15), (1, 1)))
    planes = [xp[:, ci, :, dx:dx + W] for ci in range(3) for dx in range(3)]
    planes.append(jnp.ones_like(planes[0]))
    xs = jnp.stack(planes, axis=1)
    xs = jnp.pad(xs, ((0, 0), (0, 0), (0, 0), (0, 32))).astype(jnp.bfloat16)

    wmat = conv_w.transpose(1, 3, 2, 0).reshape(27, IN_FEATS)
    wmat = jnp.concatenate(
        [wmat, conv_b[None, :], jnp.zeros((4, IN_FEATS), conv_b.dtype)], axis=0)
    wmat = wmat.astype(jnp.bfloat16)

    h = pl.pallas_call(
        _conv_mean_body,
        grid=(B,),
        in_specs=[
            pl.BlockSpec((1, 10, 240, 256), lambda i: (i, 0, 0, 0)),
            pl.BlockSpec((KPAD, IN_FEATS), lambda i: (0, 0)),
        ],
        out_specs=pl.BlockSpec((1, 1, IN_FEATS), lambda i: (i, 0, 0)),
        out_shape=jax.ShapeDtypeStruct((B, 1, IN_FEATS), jnp.float32),
        compiler_params=pltpu.CompilerParams(
            dimension_semantics=("parallel",)),
    )(xs, wmat)

    nodes = h.reshape(B * NUM_NODES, NODE_DIM)
    dvec = _deg_vector()

    aggs1 = _sc_segment_suffix_sum(nodes, NODE_DIM)
    h1 = pl.pallas_call(
        _gcn_layer1_body,
        out_shape=jax.ShapeDtypeStruct((B * NUM_NODES, HID), jnp.float32),
    )(aggs1, dvec, w1.T, b1[None, :])

    aggs2 = _sc_segment_suffix_sum(h1, 128)
    out = pl.pallas_call(
        _gcn_layer2_body,
        out_shape=jax.ShapeDtypeStruct((B, NUM_CLASSES), jnp.float32),
    )(aggs2, dvec, _pool_matrix(), w2.T, b2[None, :], wfc.T, bfc[None, :])
    return out


# speedup vs baseline: 1.1864x; 1.1864x over previous
import functools

import numpy as np
import jax
import jax.numpy as jnp
from jax.experimental import pallas as pl
from jax.experimental.pallas import tpu as pltpu

B = 8
IN_FEATS = 256
HID = 512
NUM_CLASSES = 1000
NUM_NODES = 16
NODE_DIM = IN_FEATS // NUM_NODES
H = W = 224
KPAD = 32


def _conv_mean_body(x_ref, w_ref, o_ref):
    zrows = jnp.zeros((2, 256), jnp.bfloat16)

    def block_step(i, acc):
        y0 = pl.multiple_of(i * 16, 8)
        win = [x_ref[0, j, pl.ds(y0, 24), :] for j in range(10)]
        for r in range(16):
            pt = jnp.concatenate([wj[r:r + 3] for wj in win] + [zrows],
                                 axis=0)
            z = jax.lax.dot_general(
                pt, w_ref[...],
                dimension_numbers=(((0,), (0,)), ((), ())),
                preferred_element_type=jnp.float32)
            acc = acc + jnp.sum(
                jnp.maximum(z, 0.0).reshape(32, 8, IN_FEATS), axis=0)
        return acc

    acc = jax.lax.fori_loop(0, H // 16, block_step,
                            jnp.zeros((8, IN_FEATS), jnp.float32))
    o_ref[0, 0, :] = jnp.sum(acc, axis=0) * jnp.float32(1.0 / (H * W))


def _gcn_layer1_body(agg_ref, d_ref, w1t_ref, b1_ref, o_ref):
    f32 = jnp.float32
    agg = agg_ref[...] * d_ref[...]
    o_ref[...] = jnp.maximum(
        jax.lax.dot_general(agg, w1t_ref[...], (((1,), (0,)), ((), ())),
                            preferred_element_type=f32) + b1_ref[...], 0.0)


def _gcn_layer2_body(agg_ref, d_ref, p_ref, w2t_ref, b2_ref,
                     wfct_ref, bfc_ref, o_ref):
    f32 = jnp.float32
    agg = agg_ref[...] * d_ref[...]
    h2 = jnp.maximum(
        jax.lax.dot_general(agg, w2t_ref[...], (((1,), (0,)), ((), ())),
                            preferred_element_type=f32) + b2_ref[...], 0.0)
    pooled = jax.lax.dot_general(
        p_ref[...], h2, (((1,), (0,)), ((), ())), preferred_element_type=f32)
    o_ref[...] = jax.lax.dot_general(
        pooled, wfct_ref[...], (((1,), (0,)), ((), ())),
        preferred_element_type=f32) + bfc_ref[...]


def _deg_vector():
    d = np.array([1.0 / max(NUM_NODES - 1 - i, 1) for i in range(NUM_NODES)],
                 np.float32)
    return jnp.asarray(np.tile(d, B)[:, None])


def _pool_matrix():
    p = np.kron(np.eye(B, dtype=np.float32),
                np.full((1, NUM_NODES), 1.0 / NUM_NODES, np.float32))
    return jnp.asarray(p)


def _sc_segment_suffix_sum(data, chunk_w):
    from jax.experimental.pallas import tpu_sc as plsc
    nrows, d_feat = data.shape
    d_chunks = d_feat // chunk_w
    ntasks = B * d_chunks
    mesh = plsc.VectorSubcoreMesh(core_axis_name="c", subcore_axis_name="s")

    @functools.partial(
        pl.kernel,
        out_type=jax.ShapeDtypeStruct((nrows, d_feat), jnp.float32),
        mesh=mesh,
        scratch_types=[pltpu.VMEM((16, NUM_NODES, chunk_w), jnp.float32),
                       pltpu.VMEM((16, NUM_NODES, chunk_w), jnp.float32),
                       pltpu.SemaphoreType.DMA((16,))],
    )
    def agg_kernel(x_hbm, o_hbm, in_vs, out_vs, sems):
        sidx = jax.lax.axis_index("s")
        t = jax.lax.axis_index("c") * 16 + sidx
        in_v = in_vs.at[sidx]
        out_v = out_vs.at[sidx]
        sem = sems.at[sidx]

        @pl.when(t < ntasks)
        def _():
            bq = t // d_chunks
            cq = jax.lax.rem(t, d_chunks)
            r0 = bq * NUM_NODES
            c0 = cq * chunk_w
            pltpu.async_copy(
                x_hbm.at[pl.ds(r0, NUM_NODES), pl.ds(c0, chunk_w)],
                in_v, sem).wait()

            @pl.loop(0, chunk_w, step=16)
            def _(l):
                sl = pl.ds(l, 16)
                out_v.at[NUM_NODES - 1, sl][...] = jnp.zeros((16,), jnp.float32)

                @pl.loop(0, NUM_NODES - 1)
                def _(ii):
                    i = NUM_NODES - 2 - ii
                    up = in_v.at[i + 1, sl][...]
                    out_v.at[i, sl][...] = up
                    in_v.at[i, sl][...] = in_v.at[i, sl][...] + up

            pltpu.async_copy(
                out_v, o_hbm.at[pl.ds(r0, NUM_NODES), pl.ds(c0, chunk_w)],
                sem).wait()

    return agg_kernel(data)


def kernel(x, conv_w, conv_b, w1, b1, w2, b2, wfc, bfc):
    xp = jnp.pad(x, ((0, 0), (0, 0), (1, 15), (1, 1)))
    planes = [xp[:, ci, :, dx:dx + W] for ci in range(3) for dx in range(3)]
    planes.append(jnp.ones_like(planes[0]))
    xs = jnp.stack(planes, axis=1)
    xs = jnp.pad(xs, ((0, 0), (0, 0), (0, 0), (0, 32))).astype(jnp.bfloat16)

    wmat = conv_w.transpose(1, 3, 2, 0).reshape(27, IN_FEATS)
    wmat = jnp.concatenate(
        [wmat, conv_b[None, :], jnp.zeros((4, IN_FEATS), conv_b.dtype)], axis=0)
    wmat = wmat.astype(jnp.bfloat16)

    h = pl.pallas_call(
        _conv_mean_body,
        grid=(B,),
        in_specs=[
            pl.BlockSpec((1, 10, 240, 256), lambda i: (i, 0, 0, 0)),
            pl.BlockSpec((KPAD, IN_FEATS), lambda i: (0, 0)),
        ],
        out_specs=pl.BlockSpec((1, 1, IN_FEATS), lambda i: (i, 0, 0)),
        out_shape=jax.ShapeDtypeStruct((B, 1, IN_FEATS), jnp.float32),
        compiler_params=pltpu.CompilerParams(
            dimension_semantics=("parallel",)),
    )(xs, wmat)

    nodes = h.reshape(B * NUM_NODES, NODE_DIM)
    dvec = _deg_vector()

    aggs1 = _sc_segment_suffix_sum(nodes, NODE_DIM)
    h1 = pl.pallas_call(
        _gcn_layer1_body,
        out_shape=jax.ShapeDtypeStruct((B * NUM_NODES, HID), jnp.float32),
    )(aggs1, dvec, w1.T, b1[None, :])

    aggs2 = _sc_segment_suffix_sum(h1, 128)
    out = pl.pallas_call(
        _gcn_layer2_body,
        out_shape=jax.ShapeDtypeStruct((B, NUM_CLASSES), jnp.float32),
    )(aggs2, dvec, _pool_matrix(), w2.T, b2[None, :], wfc.T, bfc[None, :])
    return out


# speedup vs baseline: 1.3049x; 1.0999x over previous
import functools

import numpy as np
import jax
import jax.numpy as jnp
from jax.experimental import pallas as pl
from jax.experimental.pallas import tpu as pltpu

B = 8
IN_FEATS = 256
HID = 512
NUM_CLASSES = 1000
NUM_NODES = 16
NODE_DIM = IN_FEATS // NUM_NODES
H = W = 224
KPAD = 32


def _conv_mean_body(x_ref, w_ref, o_ref):
    zrows = jnp.zeros((2, 256), jnp.bfloat16)

    def block_step(i, acc):
        y0 = pl.multiple_of(i * 32, 8)
        win = [x_ref[0, j, pl.ds(y0, 40), :] for j in range(10)]
        for r in range(32):
            pt = jnp.concatenate([wj[r:r + 3] for wj in win] + [zrows],
                                 axis=0)
            z = jax.lax.dot_general(
                pt, w_ref[...],
                dimension_numbers=(((0,), (0,)), ((), ())),
                preferred_element_type=jnp.float32)
            acc = acc + jnp.sum(
                jnp.maximum(z, 0.0).reshape(32, 8, IN_FEATS), axis=0)
        return acc

    acc = jax.lax.fori_loop(0, H // 32, block_step,
                            jnp.zeros((8, IN_FEATS), jnp.float32))
    o_ref[0, 0, :] = jnp.sum(acc, axis=0) * jnp.float32(1.0 / (H * W))


def _gcn_layer1_body(agg_ref, d_ref, w1t_ref, b1_ref, o_ref):
    f32 = jnp.float32
    agg = agg_ref[...] * d_ref[...]
    o_ref[...] = jnp.maximum(
        jax.lax.dot_general(agg, w1t_ref[...], (((1,), (0,)), ((), ())),
                            preferred_element_type=f32) + b1_ref[...], 0.0)


def _gcn_layer2_body(agg_ref, d_ref, p_ref, w2t_ref, b2_ref,
                     wfct_ref, bfc_ref, o_ref):
    f32 = jnp.float32
    agg = agg_ref[...] * d_ref[...]
    h2 = jnp.maximum(
        jax.lax.dot_general(agg, w2t_ref[...], (((1,), (0,)), ((), ())),
                            preferred_element_type=f32) + b2_ref[...], 0.0)
    pooled = jax.lax.dot_general(
        p_ref[...], h2, (((1,), (0,)), ((), ())), preferred_element_type=f32)
    o_ref[...] = jax.lax.dot_general(
        pooled, wfct_ref[...], (((1,), (0,)), ((), ())),
        preferred_element_type=f32) + bfc_ref[...]


def _deg_vector():
    d = np.array([1.0 / max(NUM_NODES - 1 - i, 1) for i in range(NUM_NODES)],
                 np.float32)
    return jnp.asarray(np.tile(d, B)[:, None])


def _pool_matrix():
    p = np.kron(np.eye(B, dtype=np.float32),
                np.full((1, NUM_NODES), 1.0 / NUM_NODES, np.float32))
    return jnp.asarray(p)


def _sc_segment_suffix_sum(data, chunk_w):
    from jax.experimental.pallas import tpu_sc as plsc
    nrows, d_feat = data.shape
    d_chunks = d_feat // chunk_w
    ntasks = B * d_chunks
    mesh = plsc.VectorSubcoreMesh(core_axis_name="c", subcore_axis_name="s")

    @functools.partial(
        pl.kernel,
        out_type=jax.ShapeDtypeStruct((nrows, d_feat), jnp.float32),
        mesh=mesh,
        scratch_types=[pltpu.VMEM((16, NUM_NODES, chunk_w), jnp.float32),
                       pltpu.VMEM((16, NUM_NODES, chunk_w), jnp.float32),
                       pltpu.SemaphoreType.DMA((16,))],
    )
    def agg_kernel(x_hbm, o_hbm, in_vs, out_vs, sems):
        sidx = jax.lax.axis_index("s")
        t = jax.lax.axis_index("c") * 16 + sidx
        in_v = in_vs.at[sidx]
        out_v = out_vs.at[sidx]
        sem = sems.at[sidx]

        @pl.when(t < ntasks)
        def _():
            bq = t // d_chunks
            cq = jax.lax.rem(t, d_chunks)
            r0 = bq * NUM_NODES
            c0 = cq * chunk_w
            pltpu.async_copy(
                x_hbm.at[pl.ds(r0, NUM_NODES), pl.ds(c0, chunk_w)],
                in_v, sem).wait()

            @pl.loop(0, chunk_w, step=16)
            def _(l):
                sl = pl.ds(l, 16)
                out_v.at[NUM_NODES - 1, sl][...] = jnp.zeros((16,), jnp.float32)

                @pl.loop(0, NUM_NODES - 1)
                def _(ii):
                    i = NUM_NODES - 2 - ii
                    up = in_v.at[i + 1, sl][...]
                    out_v.at[i, sl][...] = up
                    in_v.at[i, sl][...] = in_v.at[i, sl][...] + up

            pltpu.async_copy(
                out_v, o_hbm.at[pl.ds(r0, NUM_NODES), pl.ds(c0, chunk_w)],
                sem).wait()

    return agg_kernel(data)


def kernel(x, conv_w, conv_b, w1, b1, w2, b2, wfc, bfc):
    xp = jnp.pad(x, ((0, 0), (0, 0), (1, 15), (1, 1)))
    planes = [xp[:, ci, :, dx:dx + W] for ci in range(3) for dx in range(3)]
    planes.append(jnp.ones_like(planes[0]))
    xs = jnp.stack(planes, axis=1)
    xs = jnp.pad(xs, ((0, 0), (0, 0), (0, 0), (0, 32))).astype(jnp.bfloat16)

    wmat = conv_w.transpose(1, 3, 2, 0).reshape(27, IN_FEATS)
    wmat = jnp.concatenate(
        [wmat, conv_b[None, :], jnp.zeros((4, IN_FEATS), conv_b.dtype)], axis=0)
    wmat = wmat.astype(jnp.bfloat16)

    h = pl.pallas_call(
        _conv_mean_body,
        grid=(B,),
        in_specs=[
            pl.BlockSpec((1, 10, 240, 256), lambda i: (i, 0, 0, 0)),
            pl.BlockSpec((KPAD, IN_FEATS), lambda i: (0, 0)),
        ],
        out_specs=pl.BlockSpec((1, 1, IN_FEATS), lambda i: (i, 0, 0)),
        out_shape=jax.ShapeDtypeStruct((B, 1, IN_FEATS), jnp.float32),
        compiler_params=pltpu.CompilerParams(
            dimension_semantics=("parallel",)),
    )(xs, wmat)

    nodes = h.reshape(B * NUM_NODES, NODE_DIM)
    dvec = _deg_vector()

    aggs1 = _sc_segment_suffix_sum(nodes, NODE_DIM)
    h1 = pl.pallas_call(
        _gcn_layer1_body,
        out_shape=jax.ShapeDtypeStruct((B * NUM_NODES, HID), jnp.float32),
    )(aggs1, dvec, w1.T, b1[None, :])

    aggs2 = _sc_segment_suffix_sum(h1, 128)
    out = pl.pallas_call(
        _gcn_layer2_body,
        out_shape=jax.ShapeDtypeStruct((B, NUM_CLASSES), jnp.float32),
    )(aggs2, dvec, _pool_matrix(), w2.T, b2[None, :], wfc.T, bfc[None, :])
    return out


# speedup vs baseline: 1.3617x; 1.0435x over previous
import functools

import numpy as np
import jax
import jax.numpy as jnp
from jax.experimental import pallas as pl
from jax.experimental.pallas import tpu as pltpu

B = 8
IN_FEATS = 256
HID = 512
NUM_CLASSES = 1000
NUM_NODES = 16
NODE_DIM = IN_FEATS // NUM_NODES
H = W = 224
KPAD = 32


def _conv_mean_body(x_ref, w_ref, o_ref):
    zrows = jnp.zeros((2, 256), jnp.bfloat16)

    def block_step(i, acc):
        y0 = pl.multiple_of(i * 56, 8)
        win = [x_ref[0, j, pl.ds(y0, 64), :] for j in range(10)]
        for r in range(56):
            pt = jnp.concatenate([wj[r:r + 3] for wj in win] + [zrows],
                                 axis=0)
            z = jax.lax.dot_general(
                pt, w_ref[...],
                dimension_numbers=(((0,), (0,)), ((), ())),
                preferred_element_type=jnp.float32)
            acc = acc + jnp.sum(
                jnp.maximum(z, 0.0).reshape(32, 8, IN_FEATS), axis=0)
        return acc

    acc = jax.lax.fori_loop(0, H // 56, block_step,
                            jnp.zeros((8, IN_FEATS), jnp.float32))
    o_ref[0, 0, :] = jnp.sum(acc, axis=0) * jnp.float32(1.0 / (H * W))


def _gcn_layer1_body(agg_ref, d_ref, w1t_ref, b1_ref, o_ref):
    f32 = jnp.float32
    agg = agg_ref[...] * d_ref[...]
    o_ref[...] = jnp.maximum(
        jax.lax.dot_general(agg, w1t_ref[...], (((1,), (0,)), ((), ())),
                            preferred_element_type=f32) + b1_ref[...], 0.0)


def _gcn_layer2_body(agg_ref, d_ref, p_ref, w2t_ref, b2_ref,
                     wfct_ref, bfc_ref, o_ref):
    f32 = jnp.float32
    agg = agg_ref[...] * d_ref[...]
    h2 = jnp.maximum(
        jax.lax.dot_general(agg, w2t_ref[...], (((1,), (0,)), ((), ())),
                            preferred_element_type=f32) + b2_ref[...], 0.0)
    pooled = jax.lax.dot_general(
        p_ref[...], h2, (((1,), (0,)), ((), ())), preferred_element_type=f32)
    o_ref[...] = jax.lax.dot_general(
        pooled, wfct_ref[...], (((1,), (0,)), ((), ())),
        preferred_element_type=f32) + bfc_ref[...]


def _deg_vector():
    d = np.array([1.0 / max(NUM_NODES - 1 - i, 1) for i in range(NUM_NODES)],
                 np.float32)
    return jnp.asarray(np.tile(d, B)[:, None])


def _pool_matrix():
    p = np.kron(np.eye(B, dtype=np.float32),
                np.full((1, NUM_NODES), 1.0 / NUM_NODES, np.float32))
    return jnp.asarray(p)


def _sc_segment_suffix_sum(data, chunk_w):
    from jax.experimental.pallas import tpu_sc as plsc
    nrows, d_feat = data.shape
    d_chunks = d_feat // chunk_w
    ntasks = B * d_chunks
    mesh = plsc.VectorSubcoreMesh(core_axis_name="c", subcore_axis_name="s")

    @functools.partial(
        pl.kernel,
        out_type=jax.ShapeDtypeStruct((nrows, d_feat), jnp.float32),
        mesh=mesh,
        scratch_types=[pltpu.VMEM((16, NUM_NODES, chunk_w), jnp.float32),
                       pltpu.VMEM((16, NUM_NODES, chunk_w), jnp.float32),
                       pltpu.SemaphoreType.DMA((16,))],
    )
    def agg_kernel(x_hbm, o_hbm, in_vs, out_vs, sems):
        sidx = jax.lax.axis_index("s")
        t = jax.lax.axis_index("c") * 16 + sidx
        in_v = in_vs.at[sidx]
        out_v = out_vs.at[sidx]
        sem = sems.at[sidx]

        @pl.when(t < ntasks)
        def _():
            bq = t // d_chunks
            cq = jax.lax.rem(t, d_chunks)
            r0 = bq * NUM_NODES
            c0 = cq * chunk_w
            pltpu.async_copy(
                x_hbm.at[pl.ds(r0, NUM_NODES), pl.ds(c0, chunk_w)],
                in_v, sem).wait()

            @pl.loop(0, chunk_w, step=16)
            def _(l):
                sl = pl.ds(l, 16)
                out_v.at[NUM_NODES - 1, sl][...] = jnp.zeros((16,), jnp.float32)

                @pl.loop(0, NUM_NODES - 1)
                def _(ii):
                    i = NUM_NODES - 2 - ii
                    up = in_v.at[i + 1, sl][...]
                    out_v.at[i, sl][...] = up
                    in_v.at[i, sl][...] = in_v.at[i, sl][...] + up

            pltpu.async_copy(
                out_v, o_hbm.at[pl.ds(r0, NUM_NODES), pl.ds(c0, chunk_w)],
                sem).wait()

    return agg_kernel(data)


def kernel(x, conv_w, conv_b, w1, b1, w2, b2, wfc, bfc):
    xp = jnp.pad(x, ((0, 0), (0, 0), (1, 15), (1, 1)))
    planes = [xp[:, ci, :, dx:dx + W] for ci in range(3) for dx in range(3)]
    planes.append(jnp.ones_like(planes[0]))
    xs = jnp.stack(planes, axis=1)
    xs = jnp.pad(xs, ((0, 0), (0, 0), (0, 0), (0, 32))).astype(jnp.bfloat16)

    wmat = conv_w.transpose(1, 3, 2, 0).reshape(27, IN_FEATS)
    wmat = jnp.concatenate(
        [wmat, conv_b[None, :], jnp.zeros((4, IN_FEATS), conv_b.dtype)], axis=0)
    wmat = wmat.astype(jnp.bfloat16)

    h = pl.pallas_call(
        _conv_mean_body,
        grid=(B,),
        in_specs=[
            pl.BlockSpec((1, 10, 240, 256), lambda i: (i, 0, 0, 0)),
            pl.BlockSpec((KPAD, IN_FEATS), lambda i: (0, 0)),
        ],
        out_specs=pl.BlockSpec((1, 1, IN_FEATS), lambda i: (i, 0, 0)),
        out_shape=jax.ShapeDtypeStruct((B, 1, IN_FEATS), jnp.float32),
        compiler_params=pltpu.CompilerParams(
            dimension_semantics=("parallel",)),
    )(xs, wmat)

    nodes = h.reshape(B * NUM_NODES, NODE_DIM)
    dvec = _deg_vector()

    aggs1 = _sc_segment_suffix_sum(nodes, NODE_DIM)
    h1 = pl.pallas_call(
        _gcn_layer1_body,
        out_shape=jax.ShapeDtypeStruct((B * NUM_NODES, HID), jnp.float32),
    )(aggs1, dvec, w1.T, b1[None, :])

    aggs2 = _sc_segment_suffix_sum(h1, 128)
    out = pl.pallas_call(
        _gcn_layer2_body,
        out_shape=jax.ShapeDtypeStruct((B, NUM_CLASSES), jnp.float32),
    )(aggs2, dvec, _pool_matrix(), w2.T, b2[None, :], wfc.T, bfc[None, :])
    return out
